# Initial kernel scaffold; baseline (speedup 1.0000x reference)
import jax
import jax.numpy as jnp
from jax.experimental import pallas as pl


def kernel(user_feat, repo_feat, edge_src, edge_dst, pos_src, pos_dst, neg_src, neg_dst, W_user, b_user, W_repo, b_repo, W_h_u2r, b_h_u2r, W_h_r2u, b_h_r2u, W_o_u2r, b_o_u2r, W_o_r2u, b_o_r2u):
    raise NotImplementedError("write your pallas kernel here")


# speedup vs baseline: 3.7690x; 3.7690x over previous
import functools
import jax
import jax.numpy as jnp
from jax import lax
from jax.experimental import pallas as pl
from jax.experimental.pallas import tpu as pltpu
from jax.experimental.pallas import tpu_sc as plsc

_N_U = 50000
_N_R = 50000
_E = 800000
_EP = 100000
_D1, _D2, _D3 = 64, 64, 32
_NC, _NS, _L = 2, 16, 16
_CH = 128
_NCH_E = _E // _CH
_EPP = 100096
_NCH_P = _EPP // _CH
_NP = 50176
_RPT = _NP // _NS
_ZR = 224

_SC_MESH = plsc.VectorSubcoreMesh(core_axis_name="c", subcore_axis_name="s")
_SC_PARAMS = pltpu.CompilerParams(use_tc_tiling_on_sc=False)


def _fill_rows(buf, nrows, ncols, value):
  def row(i, _):
    for c0 in range(0, ncols, _L):
      buf[i, c0:c0 + _L] = jnp.full((_L,), value, jnp.float32)
    return _
  lax.fori_loop(0, nrows, row, None)


def _zero_own_rows(agg_sh, zbuf, tid):
  r0 = tid * _RPT
  def blk(k, _):
    pltpu.sync_copy(zbuf, agg_sh.at[pl.ds(r0 + k * _ZR, _ZR)])
    return _
  lax.fori_loop(0, _RPT // _ZR, blk, None)


def _flush_own_rows(agg_sh, out_hbm, tid):
  r0 = tid * _RPT
  pltpu.sync_copy(agg_sh.at[pl.ds(r0, _RPT)], out_hbm.at[pl.ds(r0, _RPT)])


def _edge_chunk_loop(tid, nch, body_fn):
  nb = (nch - tid + _NS - 1) // _NS
  def body(j, _):
    c = tid + _NS * j
    body_fn(c * _CH)
    return _
  lax.fori_loop(0, nb, body, None)


def _agg_direction(tid, tab_hbm, gidx_hbm, sidx_hbm, out_hbm,
                   idxg_v, idxs_v, rows_v, zbuf, agg_sh):
  _zero_own_rows(agg_sh, zbuf, tid)
  plsc.subcore_barrier()
  def chunk(base):
    pltpu.sync_copy(gidx_hbm.at[pl.ds(base, _CH)], idxg_v)
    pltpu.sync_copy(sidx_hbm.at[pl.ds(base, _CH)], idxs_v)
    pltpu.sync_copy(tab_hbm.at[idxg_v], rows_v)
    pltpu.sync_copy(rows_v, agg_sh.at[idxs_v], add=True)
  _edge_chunk_loop(tid, _NCH_E, chunk)
  plsc.subcore_barrier()
  _flush_own_rows(agg_sh, out_hbm, tid)


def _deg_kernel(src_hbm, dst_hbm, degu_hbm, degr_hbm, idx_v, ones_v, zbuf, agg_sh):
  core = lax.axis_index("c")
  tid = lax.axis_index("s")
  _fill_rows(ones_v, _CH, _L, 1.0)
  _fill_rows(zbuf, _ZR, _L, 0.0)
  _zero_own_rows(agg_sh, zbuf, tid)
  plsc.subcore_barrier()

  def count(eh, outh):
    def chunk(base):
      pltpu.sync_copy(eh.at[pl.ds(base, _CH)], idx_v)
      pltpu.sync_copy(ones_v, agg_sh.at[idx_v], add=True)
    _edge_chunk_loop(tid, _NCH_E, chunk)
    plsc.subcore_barrier()
    _flush_own_rows(agg_sh, outh, tid)

  @pl.when(core == 0)
  def _():
    count(src_hbm, degu_hbm)

  @pl.when(core == 1)
  def _():
    count(dst_hbm, degr_hbm)


def _sc_degrees(edge_src, edge_dst):
  out = (jax.ShapeDtypeStruct((_NP, _L), jnp.float32),
         jax.ShapeDtypeStruct((_NP, _L), jnp.float32))
  return pl.kernel(
      _deg_kernel,
      out_type=out,
      mesh=_SC_MESH,
      compiler_params=_SC_PARAMS,
      scratch_types=[
          pltpu.VMEM((_CH,), jnp.int32),
          pltpu.VMEM((_CH, _L), jnp.float32),
          pltpu.VMEM((_ZR, _L), jnp.float32),
          pltpu.VMEM_SHARED((_NP, _L), jnp.float32),
      ],
  )(edge_src, edge_dst)


def _agg64_kernel(tul, tuh, trl, trh, src_hbm, dst_hbm,
                  arl, arh, aul, auh,
                  idxg_v, idxs_v, rows_v, zbuf, agg_sh):
  core = lax.axis_index("c")
  tid = lax.axis_index("s")
  _fill_rows(zbuf, _ZR, _D3, 0.0)

  def run(tab_u2r, out_u2r, tab_r2u, out_r2u):
    _agg_direction(tid, tab_u2r, src_hbm, dst_hbm, out_u2r,
                   idxg_v, idxs_v, rows_v, zbuf, agg_sh)
    _agg_direction(tid, tab_r2u, dst_hbm, src_hbm, out_r2u,
                   idxg_v, idxs_v, rows_v, zbuf, agg_sh)

  @pl.when(core == 0)
  def _():
    run(tul, arl, trl, aul)

  @pl.when(core == 1)
  def _():
    run(tuh, arh, trh, auh)


def _sc_agg64(tul, tuh, trl, trh, edge_src, edge_dst):
  half = jax.ShapeDtypeStruct((_NP, _D3), jnp.float32)
  return pl.kernel(
      _agg64_kernel,
      out_type=(half, half, half, half),
      mesh=_SC_MESH,
      compiler_params=_SC_PARAMS,
      scratch_types=[
          pltpu.VMEM((_CH,), jnp.int32),
          pltpu.VMEM((_CH,), jnp.int32),
          pltpu.VMEM((_CH, _D3), jnp.float32),
          pltpu.VMEM((_ZR, _D3), jnp.float32),
          pltpu.VMEM_SHARED((_NP, _D3), jnp.float32),
      ],
  )(tul, tuh, trl, trh, edge_src, edge_dst)


def _agg32_kernel(qu, qr, src_hbm, dst_hbm, aggr2, aggu2,
                  idxg_v, idxs_v, rows_v, zbuf, agg_sh):
  core = lax.axis_index("c")
  tid = lax.axis_index("s")
  _fill_rows(zbuf, _ZR, _D3, 0.0)

  @pl.when(core == 0)
  def _():
    _agg_direction(tid, qu, src_hbm, dst_hbm, aggr2,
                   idxg_v, idxs_v, rows_v, zbuf, agg_sh)

  @pl.when(core == 1)
  def _():
    _agg_direction(tid, qr, dst_hbm, src_hbm, aggu2,
                   idxg_v, idxs_v, rows_v, zbuf, agg_sh)


def _sc_agg32(qu, qr, edge_src, edge_dst):
  full = jax.ShapeDtypeStruct((_NP, _D3), jnp.float32)
  return pl.kernel(
      _agg32_kernel,
      out_type=(full, full),
      mesh=_SC_MESH,
      compiler_params=_SC_PARAMS,
      scratch_types=[
          pltpu.VMEM((_CH,), jnp.int32),
          pltpu.VMEM((_CH,), jnp.int32),
          pltpu.VMEM((_CH, _D3), jnp.float32),
          pltpu.VMEM((_ZR, _D3), jnp.float32),
          pltpu.VMEM_SHARED((_NP, _D3), jnp.float32),
      ],
  )(qu, qr, edge_src, edge_dst)


def _score_kernel(nu, nr, ps, pd, ns_, nd, gap, gbp, gan, gbn, idx_v, rows_v):
  core = lax.axis_index("c")
  tid = lax.axis_index("s")

  def gth(tab_hbm, idx_hbm, out_hbm):
    def chunk(base):
      pltpu.sync_copy(idx_hbm.at[pl.ds(base, _CH)], idx_v)
      pltpu.sync_copy(tab_hbm.at[idx_v], rows_v)
      pltpu.sync_copy(rows_v, out_hbm.at[pl.ds(base, _CH)])
    _edge_chunk_loop(tid, _NCH_P, chunk)

  @pl.when(core == 0)
  def _():
    gth(nu, ps, gap)
    gth(nr, pd, gbp)

  @pl.when(core == 1)
  def _():
    gth(nu, ns_, gan)
    gth(nr, nd, gbn)


def _sc_score_gather(nu, nr, ps, pd, ns_, nd):
  g = jax.ShapeDtypeStruct((_EPP, _D3), jnp.float32)
  return pl.kernel(
      _score_kernel,
      out_type=(g, g, g, g),
      mesh=_SC_MESH,
      compiler_params=_SC_PARAMS,
      scratch_types=[
          pltpu.VMEM((_CH,), jnp.int32),
          pltpu.VMEM((_CH, _D3), jnp.float32),
      ],
  )(nu, nr, ps, pd, ns_, nd)


_RB = 2000
_GRID_N = _N_U // _RB


def _norm_col(deg_col):
  return jnp.where(deg_col > 0, lax.rsqrt(jnp.maximum(deg_col, 1.0)), 0.0)


def _full(shape):
  return pl.BlockSpec(shape, lambda i: tuple(0 for _ in shape))


def _rows(shape):
  return pl.BlockSpec(shape, lambda i: (i,) + tuple(0 for _ in shape[1:]))


def _embed_body(u_ref, r_ref, du_ref, dr_ref, wu_ref, bu_ref, wr_ref, br_ref,
                whu_ref, whr_ref, pul_ref, puh_ref, prl_ref, prh_ref):
  normu = _norm_col(du_ref[:, 0:1])
  normr = _norm_col(dr_ref[:, 0:1])
  hu = jnp.dot(u_ref[...], wu_ref[...], preferred_element_type=jnp.float32) + bu_ref[...]
  pu = jnp.dot(hu, whu_ref[...], preferred_element_type=jnp.float32) * normu
  pul_ref[...] = pu[:, :_D3]
  puh_ref[...] = pu[:, _D3:]
  hr = jnp.dot(r_ref[...], wr_ref[...], preferred_element_type=jnp.float32) + br_ref[...]
  pr = jnp.dot(hr, whr_ref[...], preferred_element_type=jnp.float32) * normr
  prl_ref[...] = pr[:, :_D3]
  prh_ref[...] = pr[:, _D3:]


def _tc_embed(user_feat, repo_feat, degu, degr, w_user, b_user, w_repo, b_repo,
              w_h_u2r, w_h_r2u):
  half = jax.ShapeDtypeStruct((_N_U, _D3), jnp.float32)
  return pl.pallas_call(
      _embed_body,
      grid=(_GRID_N,),
      in_specs=[
          _rows((_RB, 128)), _rows((_RB, 128)),
          _rows((_RB, _L)), _rows((_RB, _L)),
          _full((128, _D1)), _full((1, _D1)),
          _full((128, _D1)), _full((1, _D1)),
          _full((_D1, _D2)), _full((_D1, _D2)),
      ],
      out_specs=[_rows((_RB, _D3))] * 4,
      out_shape=(half, half, half, half),
  )(user_feat, repo_feat, degu, degr, w_user, b_user, w_repo, b_repo,
    w_h_u2r, w_h_r2u)


def _mid_body(arl_ref, arh_ref, aul_ref, auh_ref, du_ref, dr_ref,
              bhu_ref, bhr_ref, wou_ref, wor_ref, qu_ref, qr_ref):
  normu = _norm_col(du_ref[:, 0:1])
  normr = _norm_col(dr_ref[:, 0:1])
  h_user_1 = jnp.concatenate([aul_ref[...], auh_ref[...]], axis=1) * normu + bhr_ref[...]
  qu_ref[...] = jnp.dot(h_user_1, wou_ref[...], preferred_element_type=jnp.float32) * normu
  h_repo_1 = jnp.concatenate([arl_ref[...], arh_ref[...]], axis=1) * normr + bhu_ref[...]
  qr_ref[...] = jnp.dot(h_repo_1, wor_ref[...], preferred_element_type=jnp.float32) * normr


def _tc_mid(arl, arh, aul, auh, degu, degr, b_h_u2r, b_h_r2u, w_o_u2r, w_o_r2u):
  full = jax.ShapeDtypeStruct((_N_U, _D3), jnp.float32)
  return pl.pallas_call(
      _mid_body,
      grid=(_GRID_N,),
      in_specs=[
          _rows((_RB, _D3))] * 4 + [
          _rows((_RB, _L)), _rows((_RB, _L)),
          _full((1, _D2)), _full((1, _D2)),
          _full((_D2, _D3)), _full((_D2, _D3)),
      ],
      out_specs=[_rows((_RB, _D3))] * 2,
      out_shape=(full, full),
  )(arl, arh, aul, auh, degu, degr, b_h_u2r, b_h_r2u, w_o_u2r, w_o_r2u)


def _final_body(au2_ref, ar2_ref, du_ref, dr_ref, bou_ref, bor_ref,
                nu_ref, nr_ref):
  normu = _norm_col(du_ref[:, 0:1])
  normr = _norm_col(dr_ref[:, 0:1])
  ou = au2_ref[...] * normu + bor_ref[...]
  nu_ref[...] = ou / jnp.maximum(jnp.sqrt(jnp.sum(ou * ou, axis=1, keepdims=True)), 1e-12)
  orr = ar2_ref[...] * normr + bou_ref[...]
  nr_ref[...] = orr / jnp.maximum(jnp.sqrt(jnp.sum(orr * orr, axis=1, keepdims=True)), 1e-12)


def _tc_final(aggu2, aggr2, degu, degr, b_o_u2r, b_o_r2u):
  full = jax.ShapeDtypeStruct((_N_U, _D3), jnp.float32)
  return pl.pallas_call(
      _final_body,
      grid=(_GRID_N,),
      in_specs=[
          _rows((_RB, _D3)), _rows((_RB, _D3)),
          _rows((_RB, _L)), _rows((_RB, _L)),
          _full((1, _D3)), _full((1, _D3)),
      ],
      out_specs=[_rows((_RB, _D3))] * 2,
      out_shape=(full, full),
  )(aggu2, aggr2, degu, degr, b_o_u2r, b_o_r2u)


_SB = 6256


def _dots_body(ap_ref, bp_ref, an_ref, bn_ref, p_ref, n_ref):
  p_ref[...] = jnp.sum(ap_ref[...] * bp_ref[...], axis=1, keepdims=True)
  n_ref[...] = jnp.sum(an_ref[...] * bn_ref[...], axis=1, keepdims=True)


def _tc_dots(gap, gbp, gan, gbn):
  out = jax.ShapeDtypeStruct((_EPP, 1), jnp.float32)
  return pl.pallas_call(
      _dots_body,
      grid=(_EPP // _SB,),
      in_specs=[_rows((_SB, _D3))] * 4,
      out_specs=[_rows((_SB, 1))] * 2,
      out_shape=(out, out),
  )(gap, gbp, gan, gbn)


def kernel(user_feat, repo_feat, edge_src, edge_dst, pos_src, pos_dst,
           neg_src, neg_dst, W_user, b_user, W_repo, b_repo,
           W_h_u2r, b_h_u2r, W_h_r2u, b_h_r2u,
           W_o_u2r, b_o_u2r, W_o_r2u, b_o_r2u):
  degu, degr = _sc_degrees(edge_src, edge_dst)

  tul, tuh, trl, trh = _tc_embed(
      user_feat, repo_feat, degu, degr,
      W_user, b_user.reshape(1, -1), W_repo, b_repo.reshape(1, -1),
      W_h_u2r, W_h_r2u)

  arl, arh, aul, auh = _sc_agg64(tul, tuh, trl, trh, edge_src, edge_dst)

  qu, qr = _tc_mid(arl, arh, aul, auh, degu, degr,
                   b_h_u2r.reshape(1, -1), b_h_r2u.reshape(1, -1),
                   W_o_u2r, W_o_r2u)

  aggr2, aggu2 = _sc_agg32(qu, qr, edge_src, edge_dst)

  nu, nr = _tc_final(aggu2, aggr2, degu, degr,
                     b_o_u2r.reshape(1, -1), b_o_r2u.reshape(1, -1))

  pad = _EPP - _EP
  zpad = jnp.zeros((pad,), jnp.int32)
  ps = jnp.concatenate([pos_src, zpad])
  pd = jnp.concatenate([pos_dst, zpad])
  ns_ = jnp.concatenate([neg_src, zpad])
  nd = jnp.concatenate([neg_dst, zpad])

  gap, gbp, gan, gbn = _sc_score_gather(nu, nr, ps, pd, ns_, nd)
  pos, neg = _tc_dots(gap, gbp, gan, gbn)
  return pos[:_EP], neg[:_EP]


# speedup vs baseline: 6.7287x; 1.7853x over previous
import functools
import jax
import jax.numpy as jnp
from jax import lax
from jax.experimental import pallas as pl
from jax.experimental.pallas import tpu as pltpu
from jax.experimental.pallas import tpu_sc as plsc

_N_U = 50000
_N_R = 50000
_E = 800000
_EP = 100000
_D1, _D2, _D3 = 64, 64, 32
_NC, _NS, _L = 2, 16, 16
_CH = 128
_NP = 50176
_RPT = _NP // _NS
_ZR = 224
_DUMMY = _NP - 1

_EPAD = 804864
_NBE = 393
_SPAD = 104448
_NBS = 51

_SC_MESH = plsc.VectorSubcoreMesh(core_axis_name="c", subcore_axis_name="s")
_SC_PARAMS = pltpu.CompilerParams(use_tc_tiling_on_sc=False)


def _fill_rows(buf, nrows, ncols, value):
  def row(i, _):
    for c0 in range(0, ncols, _L):
      buf[i, c0:c0 + _L] = jnp.full((_L,), value, jnp.float32)
    return _
  lax.fori_loop(0, nrows, row, None)


def _zero_own_rows(agg_sh, zbuf, tid):
  r0 = tid * _RPT
  def blk(k, _):
    pltpu.sync_copy(zbuf, agg_sh.at[pl.ds(r0 + k * _ZR, _ZR)])
    return _
  lax.fori_loop(0, _RPT // _ZR, blk, None)


def _flush_own_rows(agg_sh, out_hbm, tid):
  r0 = tid * _RPT
  pltpu.sync_copy(agg_sh.at[pl.ds(r0, _RPT)], out_hbm.at[pl.ds(r0, _RPT)])


def _chunk_base(tid, j):
  return (tid + _NS * j) * _CH


def _run_pipeline(nb, step, prologue):
  prologue()
  step(0, 0, True, True, True)
  step(1, 1, False, True, True)
  step(2, 2, False, True, True)
  def body(j3, _):
    j = 3 * j3
    step(j, 0, False, True, True)
    step(j + 1, 1, False, True, True)
    step(j + 2, 2, False, True, True)
    return _
  lax.fori_loop(1, nb // 3 - 1, body, None)
  step(nb - 3, 0, False, True, True)
  step(nb - 2, 1, False, False, True)
  step(nb - 1, 2, False, False, False)


def _agg_direction(tid, tab, gih, sih, out_hbm, bufs, zbuf, agg_sh):
  ig, is_, rows, si, sg, ss = bufs

  def start_idx(j, r):
    base = _chunk_base(tid, j)
    pltpu.async_copy(gih.at[pl.ds(base, _CH)], ig[r], si[r])
    pltpu.async_copy(sih.at[pl.ds(base, _CH)], is_[r], si[r])

  def wait_idx(r):
    pltpu.make_async_copy(gih.at[pl.ds(0, _CH)], ig[r], si[r]).wait()
    pltpu.make_async_copy(sih.at[pl.ds(0, _CH)], is_[r], si[r]).wait()

  def step(j, r, first, start2, gath1):
    r1, r2 = (r + 1) % 3, (r + 2) % 3
    if not first:
      pltpu.make_async_copy(rows[r2], agg_sh.at[is_[r2]], ss[r2]).wait()
    if start2:
      start_idx(j + 2, r2)
    if gath1:
      wait_idx(r1)
      pltpu.async_copy(tab.at[ig[r1]], rows[r1], sg[r1])
    pltpu.make_async_copy(tab.at[ig[r]], rows[r], sg[r]).wait()
    pltpu.async_copy(rows[r], agg_sh.at[is_[r]], ss[r], add=True)

  def prologue():
    start_idx(0, 0)
    start_idx(1, 1)
    wait_idx(0)
    pltpu.async_copy(tab.at[ig[0]], rows[0], sg[0])

  _zero_own_rows(agg_sh, zbuf, tid)
  plsc.subcore_barrier()
  _run_pipeline(_NBE, step, prologue)
  pltpu.make_async_copy(rows[2], agg_sh.at[is_[2]], ss[2]).wait()
  plsc.subcore_barrier()
  _flush_own_rows(agg_sh, out_hbm, tid)


def _deg_kernel(srcs_hbm, dsts_hbm, degu_hbm, degr_hbm,
                i0, i1, i2, ones_v, zbuf, agg_sh, s0, s1, s2, ss0, ss1, ss2):
  core = lax.axis_index("c")
  tid = lax.axis_index("s")
  idx = (i0, i1, i2)
  si = (s0, s1, s2)
  ss = (ss0, ss1, ss2)
  _fill_rows(ones_v, _CH, _L, 1.0)
  _fill_rows(zbuf, _ZR, _L, 0.0)
  _zero_own_rows(agg_sh, zbuf, tid)
  plsc.subcore_barrier()

  def count(eh, outh):
    def step(j, r, first, start2, gath1):
      del gath1
      r2 = (r + 2) % 3
      if not first:
        pltpu.make_async_copy(ones_v, agg_sh.at[idx[r2]], ss[r2]).wait()
      if start2:
        base = _chunk_base(tid, j + 2)
        pltpu.async_copy(eh.at[pl.ds(base, _CH)], idx[r2], si[r2])
      pltpu.make_async_copy(eh.at[pl.ds(0, _CH)], idx[r], si[r]).wait()
      pltpu.async_copy(ones_v, agg_sh.at[idx[r]], ss[r], add=True)

    def prologue():
      pltpu.async_copy(eh.at[pl.ds(_chunk_base(tid, 0), _CH)], idx[0], si[0])
      pltpu.async_copy(eh.at[pl.ds(_chunk_base(tid, 1), _CH)], idx[1], si[1])

    _run_pipeline(_NBE, step, prologue)
    pltpu.make_async_copy(ones_v, agg_sh.at[idx[2]], ss[2]).wait()
    plsc.subcore_barrier()
    _flush_own_rows(agg_sh, outh, tid)

  @pl.when(core == 0)
  def _():
    count(srcs_hbm, degu_hbm)

  @pl.when(core == 1)
  def _():
    count(dsts_hbm, degr_hbm)


def _sc_degrees(es_s, ed_s):
  out = (jax.ShapeDtypeStruct((_NP, _L), jnp.float32),
         jax.ShapeDtypeStruct((_NP, _L), jnp.float32))
  return pl.kernel(
      _deg_kernel,
      out_type=out,
      mesh=_SC_MESH,
      compiler_params=_SC_PARAMS,
      scratch_types=[
          pltpu.VMEM((_CH,), jnp.int32),
          pltpu.VMEM((_CH,), jnp.int32),
          pltpu.VMEM((_CH,), jnp.int32),
          pltpu.VMEM((_CH, _L), jnp.float32),
          pltpu.VMEM((_ZR, _L), jnp.float32),
          pltpu.VMEM_SHARED((_NP, _L), jnp.float32),
      ] + [pltpu.SemaphoreType.DMA] * 6,
  )(es_s, ed_s)


def _agg64_kernel(tul, tuh, trl, trh, es_g, es_s, ed_g, ed_s,
                  arl, arh, aul, auh, *sc):
  core = lax.axis_index("c")
  tid = lax.axis_index("s")
  (g0, g1, g2, x0, x1, x2, r0, r1, r2, zbuf, agg_sh,
   a0, a1, a2, b0, b1, b2, c0, c1, c2) = sc
  bufs = ((g0, g1, g2), (x0, x1, x2), (r0, r1, r2),
          (a0, a1, a2), (b0, b1, b2), (c0, c1, c2))
  _fill_rows(zbuf, _ZR, _D3, 0.0)

  def run(tab_u2r, out_u2r, tab_r2u, out_r2u):
    _agg_direction(tid, tab_u2r, es_g, ed_s, out_u2r, bufs, zbuf, agg_sh)
    _agg_direction(tid, tab_r2u, ed_g, es_s, out_r2u, bufs, zbuf, agg_sh)

  @pl.when(core == 0)
  def _():
    run(tul, arl, trl, aul)

  @pl.when(core == 1)
  def _():
    run(tuh, arh, trh, auh)


def _sc_agg64(tul, tuh, trl, trh, es_g, es_s, ed_g, ed_s):
  half = jax.ShapeDtypeStruct((_NP, _D3), jnp.float32)
  return pl.kernel(
      _agg64_kernel,
      out_type=(half, half, half, half),
      mesh=_SC_MESH,
      compiler_params=_SC_PARAMS,
      scratch_types=[pltpu.VMEM((_CH,), jnp.int32)] * 6 +
                    [pltpu.VMEM((_CH, _D3), jnp.float32)] * 3 +
                    [pltpu.VMEM((_ZR, _D3), jnp.float32),
                     pltpu.VMEM_SHARED((_NP, _D3), jnp.float32)] +
                    [pltpu.SemaphoreType.DMA] * 9,
  )(tul, tuh, trl, trh, es_g, es_s, ed_g, ed_s)


def _agg32_kernel(qu, qr, es_g, es_s, ed_g, ed_s, aggr2, aggu2, *sc):
  core = lax.axis_index("c")
  tid = lax.axis_index("s")
  (g0, g1, g2, x0, x1, x2, r0, r1, r2, zbuf, agg_sh,
   a0, a1, a2, b0, b1, b2, c0, c1, c2) = sc
  bufs = ((g0, g1, g2), (x0, x1, x2), (r0, r1, r2),
          (a0, a1, a2), (b0, b1, b2), (c0, c1, c2))
  _fill_rows(zbuf, _ZR, _D3, 0.0)

  @pl.when(core == 0)
  def _():
    _agg_direction(tid, qu, es_g, ed_s, aggr2, bufs, zbuf, agg_sh)

  @pl.when(core == 1)
  def _():
    _agg_direction(tid, qr, ed_g, es_s, aggu2, bufs, zbuf, agg_sh)


def _sc_agg32(qu, qr, es_g, es_s, ed_g, ed_s):
  full = jax.ShapeDtypeStruct((_NP, _D3), jnp.float32)
  return pl.kernel(
      _agg32_kernel,
      out_type=(full, full),
      mesh=_SC_MESH,
      compiler_params=_SC_PARAMS,
      scratch_types=[pltpu.VMEM((_CH,), jnp.int32)] * 6 +
                    [pltpu.VMEM((_CH, _D3), jnp.float32)] * 3 +
                    [pltpu.VMEM((_ZR, _D3), jnp.float32),
                     pltpu.VMEM_SHARED((_NP, _D3), jnp.float32)] +
                    [pltpu.SemaphoreType.DMA] * 9,
  )(qu, qr, es_g, es_s, ed_g, ed_s)


def _score_kernel(nu, nr, ps, pd, ns_, nd, gap, gbp, gan, gbn, *sc):
  core = lax.axis_index("c")
  tid = lax.axis_index("s")
  (g0, g1, g2, r0, r1, r2, a0, a1, a2, b0, b1, b2, c0, c1, c2) = sc
  ig = (g0, g1, g2)
  rows = (r0, r1, r2)
  si = (a0, a1, a2)
  sg = (b0, b1, b2)
  so = (c0, c1, c2)

  def gth(tab, idx_hbm, out_hbm):
    def step(j, r, first, start2, gath1):
      r1, r2 = (r + 1) % 3, (r + 2) % 3
      if not first:
        pltpu.make_async_copy(rows[r2], out_hbm.at[pl.ds(0, _CH)], so[r2]).wait()
      if start2:
        base = _chunk_base(tid, j + 2)
        pltpu.async_copy(idx_hbm.at[pl.ds(base, _CH)], ig[r2], si[r2])
      if gath1:
        pltpu.make_async_copy(idx_hbm.at[pl.ds(0, _CH)], ig[r1], si[r1]).wait()
        pltpu.async_copy(tab.at[ig[r1]], rows[r1], sg[r1])
      pltpu.make_async_copy(tab.at[ig[r]], rows[r], sg[r]).wait()
      pltpu.async_copy(rows[r], out_hbm.at[pl.ds(_chunk_base(tid, j), _CH)], so[r])

    def prologue():
      pltpu.async_copy(idx_hbm.at[pl.ds(_chunk_base(tid, 0), _CH)], ig[0], si[0])
      pltpu.async_copy(idx_hbm.at[pl.ds(_chunk_base(tid, 1), _CH)], ig[1], si[1])
      pltpu.make_async_copy(idx_hbm.at[pl.ds(0, _CH)], ig[0], si[0]).wait()
      pltpu.async_copy(tab.at[ig[0]], rows[0], sg[0])

    _run_pipeline(_NBS, step, prologue)
    pltpu.make_async_copy(rows[2], out_hbm.at[pl.ds(0, _CH)], so[2]).wait()

  @pl.when(core == 0)
  def _():
    gth(nu, ps, gap)
    gth(nr, pd, gbp)

  @pl.when(core == 1)
  def _():
    gth(nu, ns_, gan)
    gth(nr, nd, gbn)


def _sc_score_gather(nu, nr, ps, pd, ns_, nd):
  g = jax.ShapeDtypeStruct((_SPAD, _D3), jnp.float32)
  return pl.kernel(
      _score_kernel,
      out_type=(g, g, g, g),
      mesh=_SC_MESH,
      compiler_params=_SC_PARAMS,
      scratch_types=[pltpu.VMEM((_CH,), jnp.int32)] * 3 +
                    [pltpu.VMEM((_CH, _D3), jnp.float32)] * 3 +
                    [pltpu.SemaphoreType.DMA] * 9,
  )(nu, nr, ps, pd, ns_, nd)


_RB = 2000
_GRID_N = _N_U // _RB


def _norm_col(deg_col):
  return jnp.where(deg_col > 0, lax.rsqrt(jnp.maximum(deg_col, 1.0)), 0.0)


def _full(shape):
  return pl.BlockSpec(shape, lambda i: tuple(0 for _ in shape))


def _rows(shape):
  return pl.BlockSpec(shape, lambda i: (i,) + tuple(0 for _ in shape[1:]))


def _embed_body(u_ref, r_ref, du_ref, dr_ref, wu_ref, bu_ref, wr_ref, br_ref,
                whu_ref, whr_ref, pul_ref, puh_ref, prl_ref, prh_ref):
  normu = _norm_col(du_ref[:, 0:1])
  normr = _norm_col(dr_ref[:, 0:1])
  hu = jnp.dot(u_ref[...], wu_ref[...], preferred_element_type=jnp.float32) + bu_ref[...]
  pu = jnp.dot(hu, whu_ref[...], preferred_element_type=jnp.float32) * normu
  pul_ref[...] = pu[:, :_D3]
  puh_ref[...] = pu[:, _D3:]
  hr = jnp.dot(r_ref[...], wr_ref[...], preferred_element_type=jnp.float32) + br_ref[...]
  pr = jnp.dot(hr, whr_ref[...], preferred_element_type=jnp.float32) * normr
  prl_ref[...] = pr[:, :_D3]
  prh_ref[...] = pr[:, _D3:]


def _tc_embed(user_feat, repo_feat, degu, degr, w_user, b_user, w_repo, b_repo,
              w_h_u2r, w_h_r2u):
  half = jax.ShapeDtypeStruct((_N_U, _D3), jnp.float32)
  return pl.pallas_call(
      _embed_body,
      grid=(_GRID_N,),
      in_specs=[
          _rows((_RB, 128)), _rows((_RB, 128)),
          _rows((_RB, _L)), _rows((_RB, _L)),
          _full((128, _D1)), _full((1, _D1)),
          _full((128, _D1)), _full((1, _D1)),
          _full((_D1, _D2)), _full((_D1, _D2)),
      ],
      out_specs=[_rows((_RB, _D3))] * 4,
      out_shape=(half, half, half, half),
  )(user_feat, repo_feat, degu, degr, w_user, b_user, w_repo, b_repo,
    w_h_u2r, w_h_r2u)


def _mid_body(arl_ref, arh_ref, aul_ref, auh_ref, du_ref, dr_ref,
              bhu_ref, bhr_ref, wou_ref, wor_ref, qu_ref, qr_ref):
  normu = _norm_col(du_ref[:, 0:1])
  normr = _norm_col(dr_ref[:, 0:1])
  h_user_1 = jnp.concatenate([aul_ref[...], auh_ref[...]], axis=1) * normu + bhr_ref[...]
  qu_ref[...] = jnp.dot(h_user_1, wou_ref[...], preferred_element_type=jnp.float32) * normu
  h_repo_1 = jnp.concatenate([arl_ref[...], arh_ref[...]], axis=1) * normr + bhu_ref[...]
  qr_ref[...] = jnp.dot(h_repo_1, wor_ref[...], preferred_element_type=jnp.float32) * normr


def _tc_mid(arl, arh, aul, auh, degu, degr, b_h_u2r, b_h_r2u, w_o_u2r, w_o_r2u):
  full = jax.ShapeDtypeStruct((_N_U, _D3), jnp.float32)
  return pl.pallas_call(
      _mid_body,
      grid=(_GRID_N,),
      in_specs=[
          _rows((_RB, _D3))] * 4 + [
          _rows((_RB, _L)), _rows((_RB, _L)),
          _full((1, _D2)), _full((1, _D2)),
          _full((_D2, _D3)), _full((_D2, _D3)),
      ],
      out_specs=[_rows((_RB, _D3))] * 2,
      out_shape=(full, full),
  )(arl, arh, aul, auh, degu, degr, b_h_u2r, b_h_r2u, w_o_u2r, w_o_r2u)


def _final_body(au2_ref, ar2_ref, du_ref, dr_ref, bou_ref, bor_ref,
                nu_ref, nr_ref):
  normu = _norm_col(du_ref[:, 0:1])
  normr = _norm_col(dr_ref[:, 0:1])
  ou = au2_ref[...] * normu + bor_ref[...]
  nu_ref[...] = ou / jnp.maximum(jnp.sqrt(jnp.sum(ou * ou, axis=1, keepdims=True)), 1e-12)
  orr = ar2_ref[...] * normr + bou_ref[...]
  nr_ref[...] = orr / jnp.maximum(jnp.sqrt(jnp.sum(orr * orr, axis=1, keepdims=True)), 1e-12)


def _tc_final(aggu2, aggr2, degu, degr, b_o_u2r, b_o_r2u):
  full = jax.ShapeDtypeStruct((_N_U, _D3), jnp.float32)
  return pl.pallas_call(
      _final_body,
      grid=(_GRID_N,),
      in_specs=[
          _rows((_RB, _D3)), _rows((_RB, _D3)),
          _rows((_RB, _L)), _rows((_RB, _L)),
          _full((1, _D3)), _full((1, _D3)),
      ],
      out_specs=[_rows((_RB, _D3))] * 2,
      out_shape=(full, full),
  )(aggu2, aggr2, degu, degr, b_o_u2r, b_o_r2u)


_SB = _SPAD // 16


def _dots_body(ap_ref, bp_ref, an_ref, bn_ref, p_ref, n_ref):
  p_ref[...] = jnp.sum(ap_ref[...] * bp_ref[...], axis=1, keepdims=True)
  n_ref[...] = jnp.sum(an_ref[...] * bn_ref[...], axis=1, keepdims=True)


def _tc_dots(gap, gbp, gan, gbn):
  out = jax.ShapeDtypeStruct((_SPAD, 1), jnp.float32)
  return pl.pallas_call(
      _dots_body,
      grid=(_SPAD // _SB,),
      in_specs=[_rows((_SB, _D3))] * 4,
      out_specs=[_rows((_SB, 1))] * 2,
      out_shape=(out, out),
  )(gap, gbp, gan, gbn)


def kernel(user_feat, repo_feat, edge_src, edge_dst, pos_src, pos_dst,
           neg_src, neg_dst, W_user, b_user, W_repo, b_repo,
           W_h_u2r, b_h_u2r, W_h_r2u, b_h_r2u,
           W_o_u2r, b_o_u2r, W_o_r2u, b_o_r2u):
  epad = _EPAD - _E
  zer = jnp.zeros((epad,), jnp.int32)
  dum = jnp.full((epad,), _DUMMY, jnp.int32)
  es_g = jnp.concatenate([edge_src, zer])
  es_s = jnp.concatenate([edge_src, dum])
  ed_g = jnp.concatenate([edge_dst, zer])
  ed_s = jnp.concatenate([edge_dst, dum])

  degu, degr = _sc_degrees(es_s, ed_s)

  tul, tuh, trl, trh = _tc_embed(
      user_feat, repo_feat, degu, degr,
      W_user, b_user.reshape(1, -1), W_repo, b_repo.reshape(1, -1),
      W_h_u2r, W_h_r2u)

  arl, arh, aul, auh = _sc_agg64(tul, tuh, trl, trh, es_g, es_s, ed_g, ed_s)

  qu, qr = _tc_mid(arl, arh, aul, auh, degu, degr,
                   b_h_u2r.reshape(1, -1), b_h_r2u.reshape(1, -1),
                   W_o_u2r, W_o_r2u)

  aggr2, aggu2 = _sc_agg32(qu, qr, es_g, es_s, ed_g, ed_s)

  nu, nr = _tc_final(aggu2, aggr2, degu, degr,
                     b_o_u2r.reshape(1, -1), b_o_r2u.reshape(1, -1))

  spad = _SPAD - _EP
  zpad = jnp.zeros((spad,), jnp.int32)
  ps = jnp.concatenate([pos_src, zpad])
  pd = jnp.concatenate([pos_dst, zpad])
  ns_ = jnp.concatenate([neg_src, zpad])
  nd = jnp.concatenate([neg_dst, zpad])

  gap, gbp, gan, gbn = _sc_score_gather(nu, nr, ps, pd, ns_, nd)
  pos, neg = _tc_dots(gap, gbp, gan, gbn)
  return pos[:_EP], neg[:_EP]


# speedup vs baseline: 6.9898x; 1.0388x over previous
import functools
import jax
import jax.numpy as jnp
from jax import lax
from jax.experimental import pallas as pl
from jax.experimental.pallas import tpu as pltpu
from jax.experimental.pallas import tpu_sc as plsc

_N_U = 50000
_N_R = 50000
_E = 800000
_EP = 100000
_D1, _D2, _D3 = 64, 64, 32
_NC, _NS, _L = 2, 16, 16
_CH = 128
_NP = 50176
_RPT = _NP // _NS
_ZR = 224
_DUMMY = _NP - 1

_EPAD = 802816
_NBE = 392
_SPAD = 106496
_NBS = 52
_R = 4

_SC_MESH = plsc.VectorSubcoreMesh(core_axis_name="c", subcore_axis_name="s")
_SC_PARAMS = pltpu.CompilerParams(use_tc_tiling_on_sc=False)


def _fill_rows(buf, nrows, ncols, value):
  def row(i, _):
    for c0 in range(0, ncols, _L):
      buf[i, c0:c0 + _L] = jnp.full((_L,), value, jnp.float32)
    return _
  lax.fori_loop(0, nrows, row, None)


def _zero_own_rows(agg_sh, zbuf, tid):
  r0 = tid * _RPT
  def blk(k, _):
    pltpu.sync_copy(zbuf, agg_sh.at[pl.ds(r0 + k * _ZR, _ZR)])
    return _
  lax.fori_loop(0, _RPT // _ZR, blk, None)


def _flush_own_rows(agg_sh, out_hbm, tid):
  r0 = tid * _RPT
  pltpu.sync_copy(agg_sh.at[pl.ds(r0, _RPT)], out_hbm.at[pl.ds(r0, _RPT)])


def _chunk_base(tid, j):
  return (tid + _NS * j) * _CH


def _run_pipeline(nb, step, prologue):
  prologue()
  for r in range(_R):
    step(r, r, r < 2, True, True)
  def body(jr, _):
    j = _R * jr
    for r in range(_R):
      step(j + r, r, False, True, True)
    return _
  lax.fori_loop(1, nb // _R - 1, body, None)
  jb = nb - _R
  for r in range(_R):
    j = jb + r
    step(j, r, False, j + 2 < nb, j + 1 < nb)


def _agg_direction(tid, tab, gih, sih, out_hbm, bufs, zbuf, agg_sh):
  ig, is_, rows, si, sg, ss = bufs

  def start_idx(j, r):
    base = _chunk_base(tid, j)
    pltpu.async_copy(gih.at[pl.ds(base, _CH)], ig[r], si[r])
    pltpu.async_copy(sih.at[pl.ds(base, _CH)], is_[r], si[r])

  def wait_idx(r):
    pltpu.make_async_copy(gih.at[pl.ds(0, _CH)], ig[r], si[r]).wait()
    pltpu.make_async_copy(sih.at[pl.ds(0, _CH)], is_[r], si[r]).wait()

  def step(j, r, first, start2, gath1):
    r1, r2 = (r + 1) % _R, (r + 2) % _R
    if not first:
      pltpu.make_async_copy(rows[r2], agg_sh.at[is_[r2]], ss[r2]).wait()
    if start2:
      start_idx(j + 2, r2)
    if gath1:
      wait_idx(r1)
      pltpu.async_copy(tab.at[ig[r1]], rows[r1], sg[r1])
    pltpu.make_async_copy(tab.at[ig[r]], rows[r], sg[r]).wait()
    pltpu.async_copy(rows[r], agg_sh.at[is_[r]], ss[r], add=True)

  def prologue():
    start_idx(0, 0)
    start_idx(1, 1)
    wait_idx(0)
    pltpu.async_copy(tab.at[ig[0]], rows[0], sg[0])

  _zero_own_rows(agg_sh, zbuf, tid)
  plsc.subcore_barrier()
  _run_pipeline(_NBE, step, prologue)
  for rr in ((_NBE - 2) % _R, (_NBE - 1) % _R):
    pltpu.make_async_copy(rows[rr], agg_sh.at[is_[rr]], ss[rr]).wait()
  plsc.subcore_barrier()
  _flush_own_rows(agg_sh, out_hbm, tid)


def _deg_kernel(srcs_hbm, dsts_hbm, degu_hbm, degr_hbm,
                i0, i1, i2, i3, ones_v, zbuf, agg_sh,
                s0, s1, s2, s3, ss0, ss1, ss2, ss3):
  core = lax.axis_index("c")
  tid = lax.axis_index("s")
  idx = (i0, i1, i2, i3)
  si = (s0, s1, s2, s3)
  ss = (ss0, ss1, ss2, ss3)
  _fill_rows(ones_v, _CH, _L, 1.0)
  _fill_rows(zbuf, _ZR, _L, 0.0)
  _zero_own_rows(agg_sh, zbuf, tid)
  plsc.subcore_barrier()

  def count(eh, outh):
    def step(j, r, first, start2, gath1):
      del gath1
      r2 = (r + 2) % _R
      if not first:
        pltpu.make_async_copy(ones_v, agg_sh.at[idx[r2]], ss[r2]).wait()
      if start2:
        base = _chunk_base(tid, j + 2)
        pltpu.async_copy(eh.at[pl.ds(base, _CH)], idx[r2], si[r2])
      pltpu.make_async_copy(eh.at[pl.ds(0, _CH)], idx[r], si[r]).wait()
      pltpu.async_copy(ones_v, agg_sh.at[idx[r]], ss[r], add=True)

    def prologue():
      pltpu.async_copy(eh.at[pl.ds(_chunk_base(tid, 0), _CH)], idx[0], si[0])
      pltpu.async_copy(eh.at[pl.ds(_chunk_base(tid, 1), _CH)], idx[1], si[1])

    _run_pipeline(_NBE, step, prologue)
    for rr in ((_NBE - 2) % _R, (_NBE - 1) % _R):
      pltpu.make_async_copy(ones_v, agg_sh.at[idx[rr]], ss[rr]).wait()
    plsc.subcore_barrier()
    _flush_own_rows(agg_sh, outh, tid)

  @pl.when(core == 0)
  def _():
    count(srcs_hbm, degu_hbm)

  @pl.when(core == 1)
  def _():
    count(dsts_hbm, degr_hbm)


def _sc_degrees(es_s, ed_s):
  out = (jax.ShapeDtypeStruct((_NP, _L), jnp.float32),
         jax.ShapeDtypeStruct((_NP, _L), jnp.float32))
  return pl.kernel(
      _deg_kernel,
      out_type=out,
      mesh=_SC_MESH,
      compiler_params=_SC_PARAMS,
      scratch_types=[pltpu.VMEM((_CH,), jnp.int32)] * 4 + [
          pltpu.VMEM((_CH, _L), jnp.float32),
          pltpu.VMEM((_ZR, _L), jnp.float32),
          pltpu.VMEM_SHARED((_NP, _L), jnp.float32),
      ] + [pltpu.SemaphoreType.DMA] * 8,
  )(es_s, ed_s)


def _agg64_kernel(tul, tuh, trl, trh, es_g, es_s, ed_g, ed_s,
                  arl, arh, aul, auh, *sc):
  core = lax.axis_index("c")
  tid = lax.axis_index("s")
  (g0, g1, g2, g3, x0, x1, x2, x3, r0, r1, r2, r3, zbuf, agg_sh,
   a0, a1, a2, a3, b0, b1, b2, b3, c0, c1, c2, c3) = sc
  bufs = ((g0, g1, g2, g3), (x0, x1, x2, x3), (r0, r1, r2, r3),
          (a0, a1, a2, a3), (b0, b1, b2, b3), (c0, c1, c2, c3))
  _fill_rows(zbuf, _ZR, _D3, 0.0)

  def run(tab_u2r, out_u2r, tab_r2u, out_r2u):
    _agg_direction(tid, tab_u2r, es_g, ed_s, out_u2r, bufs, zbuf, agg_sh)
    _agg_direction(tid, tab_r2u, ed_g, es_s, out_r2u, bufs, zbuf, agg_sh)

  @pl.when(core == 0)
  def _():
    run(tul, arl, trl, aul)

  @pl.when(core == 1)
  def _():
    run(tuh, arh, trh, auh)


def _sc_agg64(tul, tuh, trl, trh, es_g, es_s, ed_g, ed_s):
  half = jax.ShapeDtypeStruct((_NP, _D3), jnp.float32)
  return pl.kernel(
      _agg64_kernel,
      out_type=(half, half, half, half),
      mesh=_SC_MESH,
      compiler_params=_SC_PARAMS,
      scratch_types=[pltpu.VMEM((_CH,), jnp.int32)] * 8 +
                    [pltpu.VMEM((_CH, _D3), jnp.float32)] * 4 +
                    [pltpu.VMEM((_ZR, _D3), jnp.float32),
                     pltpu.VMEM_SHARED((_NP, _D3), jnp.float32)] +
                    [pltpu.SemaphoreType.DMA] * 12,
  )(tul, tuh, trl, trh, es_g, es_s, ed_g, ed_s)


def _agg32_kernel(qu, qr, es_g, es_s, ed_g, ed_s, aggr2, aggu2, *sc):
  core = lax.axis_index("c")
  tid = lax.axis_index("s")
  (g0, g1, g2, g3, x0, x1, x2, x3, r0, r1, r2, r3, zbuf, agg_sh,
   a0, a1, a2, a3, b0, b1, b2, b3, c0, c1, c2, c3) = sc
  bufs = ((g0, g1, g2, g3), (x0, x1, x2, x3), (r0, r1, r2, r3),
          (a0, a1, a2, a3), (b0, b1, b2, b3), (c0, c1, c2, c3))
  _fill_rows(zbuf, _ZR, _D3, 0.0)

  @pl.when(core == 0)
  def _():
    _agg_direction(tid, qu, es_g, ed_s, aggr2, bufs, zbuf, agg_sh)

  @pl.when(core == 1)
  def _():
    _agg_direction(tid, qr, ed_g, es_s, aggu2, bufs, zbuf, agg_sh)


def _sc_agg32(qu, qr, es_g, es_s, ed_g, ed_s):
  full = jax.ShapeDtypeStruct((_NP, _D3), jnp.float32)
  return pl.kernel(
      _agg32_kernel,
      out_type=(full, full),
      mesh=_SC_MESH,
      compiler_params=_SC_PARAMS,
      scratch_types=[pltpu.VMEM((_CH,), jnp.int32)] * 8 +
                    [pltpu.VMEM((_CH, _D3), jnp.float32)] * 4 +
                    [pltpu.VMEM((_ZR, _D3), jnp.float32),
                     pltpu.VMEM_SHARED((_NP, _D3), jnp.float32)] +
                    [pltpu.SemaphoreType.DMA] * 12,
  )(qu, qr, es_g, es_s, ed_g, ed_s)


def _score_kernel(nu, nr, ps, pd, ns_, nd, gap, gbp, gan, gbn, *sc):
  core = lax.axis_index("c")
  tid = lax.axis_index("s")
  (g0, g1, g2, g3, r0, r1, r2, r3,
   a0, a1, a2, a3, b0, b1, b2, b3, c0, c1, c2, c3) = sc
  ig = (g0, g1, g2, g3)
  rows = (r0, r1, r2, r3)
  si = (a0, a1, a2, a3)
  sg = (b0, b1, b2, b3)
  so = (c0, c1, c2, c3)

  def gth(tab, idx_hbm, out_hbm):
    def step(j, r, first, start2, gath1):
      r1, r2 = (r + 1) % _R, (r + 2) % _R
      if not first:
        pltpu.make_async_copy(rows[r2], out_hbm.at[pl.ds(0, _CH)], so[r2]).wait()
      if start2:
        base = _chunk_base(tid, j + 2)
        pltpu.async_copy(idx_hbm.at[pl.ds(base, _CH)], ig[r2], si[r2])
      if gath1:
        pltpu.make_async_copy(idx_hbm.at[pl.ds(0, _CH)], ig[r1], si[r1]).wait()
        pltpu.async_copy(tab.at[ig[r1]], rows[r1], sg[r1])
      pltpu.make_async_copy(tab.at[ig[r]], rows[r], sg[r]).wait()
      pltpu.async_copy(rows[r], out_hbm.at[pl.ds(_chunk_base(tid, j), _CH)], so[r])

    def prologue():
      pltpu.async_copy(idx_hbm.at[pl.ds(_chunk_base(tid, 0), _CH)], ig[0], si[0])
      pltpu.async_copy(idx_hbm.at[pl.ds(_chunk_base(tid, 1), _CH)], ig[1], si[1])
      pltpu.make_async_copy(idx_hbm.at[pl.ds(0, _CH)], ig[0], si[0]).wait()
      pltpu.async_copy(tab.at[ig[0]], rows[0], sg[0])

    _run_pipeline(_NBS, step, prologue)
    for rr in ((_NBS - 2) % _R, (_NBS - 1) % _R):
      pltpu.make_async_copy(rows[rr], out_hbm.at[pl.ds(0, _CH)], so[rr]).wait()

  @pl.when(core == 0)
  def _():
    gth(nu, ps, gap)
    gth(nr, pd, gbp)

  @pl.when(core == 1)
  def _():
    gth(nu, ns_, gan)
    gth(nr, nd, gbn)


def _sc_score_gather(nu, nr, ps, pd, ns_, nd):
  g = jax.ShapeDtypeStruct((_SPAD, _D3), jnp.float32)
  return pl.kernel(
      _score_kernel,
      out_type=(g, g, g, g),
      mesh=_SC_MESH,
      compiler_params=_SC_PARAMS,
      scratch_types=[pltpu.VMEM((_CH,), jnp.int32)] * 4 +
                    [pltpu.VMEM((_CH, _D3), jnp.float32)] * 4 +
                    [pltpu.SemaphoreType.DMA] * 12,
  )(nu, nr, ps, pd, ns_, nd)


_RB = 2000
_GRID_N = _N_U // _RB


def _norm_col(deg_col):
  return jnp.where(deg_col > 0, lax.rsqrt(jnp.maximum(deg_col, 1.0)), 0.0)


def _full(shape):
  return pl.BlockSpec(shape, lambda i: tuple(0 for _ in shape))


def _rows(shape):
  return pl.BlockSpec(shape, lambda i: (i,) + tuple(0 for _ in shape[1:]))


def _embed_body(u_ref, r_ref, du_ref, dr_ref, wu_ref, bu_ref, wr_ref, br_ref,
                whu_ref, whr_ref, pul_ref, puh_ref, prl_ref, prh_ref):
  normu = _norm_col(du_ref[:, 0:1])
  normr = _norm_col(dr_ref[:, 0:1])
  hu = jnp.dot(u_ref[...], wu_ref[...], preferred_element_type=jnp.float32) + bu_ref[...]
  pu = jnp.dot(hu, whu_ref[...], preferred_element_type=jnp.float32) * normu
  pul_ref[...] = pu[:, :_D3]
  puh_ref[...] = pu[:, _D3:]
  hr = jnp.dot(r_ref[...], wr_ref[...], preferred_element_type=jnp.float32) + br_ref[...]
  pr = jnp.dot(hr, whr_ref[...], preferred_element_type=jnp.float32) * normr
  prl_ref[...] = pr[:, :_D3]
  prh_ref[...] = pr[:, _D3:]


def _tc_embed(user_feat, repo_feat, degu, degr, w_user, b_user, w_repo, b_repo,
              w_h_u2r, w_h_r2u):
  half = jax.ShapeDtypeStruct((_N_U, _D3), jnp.float32)
  return pl.pallas_call(
      _embed_body,
      grid=(_GRID_N,),
      in_specs=[
          _rows((_RB, 128)), _rows((_RB, 128)),
          _rows((_RB, _L)), _rows((_RB, _L)),
          _full((128, _D1)), _full((1, _D1)),
          _full((128, _D1)), _full((1, _D1)),
          _full((_D1, _D2)), _full((_D1, _D2)),
      ],
      out_specs=[_rows((_RB, _D3))] * 4,
      out_shape=(half, half, half, half),
  )(user_feat, repo_feat, degu, degr, w_user, b_user, w_repo, b_repo,
    w_h_u2r, w_h_r2u)


def _mid_body(arl_ref, arh_ref, aul_ref, auh_ref, du_ref, dr_ref,
              bhu_ref, bhr_ref, wou_ref, wor_ref, qu_ref, qr_ref):
  normu = _norm_col(du_ref[:, 0:1])
  normr = _norm_col(dr_ref[:, 0:1])
  h_user_1 = jnp.concatenate([aul_ref[...], auh_ref[...]], axis=1) * normu + bhr_ref[...]
  qu_ref[...] = jnp.dot(h_user_1, wou_ref[...], preferred_element_type=jnp.float32) * normu
  h_repo_1 = jnp.concatenate([arl_ref[...], arh_ref[...]], axis=1) * normr + bhu_ref[...]
  qr_ref[...] = jnp.dot(h_repo_1, wor_ref[...], preferred_element_type=jnp.float32) * normr


def _tc_mid(arl, arh, aul, auh, degu, degr, b_h_u2r, b_h_r2u, w_o_u2r, w_o_r2u):
  full = jax.ShapeDtypeStruct((_N_U, _D3), jnp.float32)
  return pl.pallas_call(
      _mid_body,
      grid=(_GRID_N,),
      in_specs=[
          _rows((_RB, _D3))] * 4 + [
          _rows((_RB, _L)), _rows((_RB, _L)),
          _full((1, _D2)), _full((1, _D2)),
          _full((_D2, _D3)), _full((_D2, _D3)),
      ],
      out_specs=[_rows((_RB, _D3))] * 2,
      out_shape=(full, full),
  )(arl, arh, aul, auh, degu, degr, b_h_u2r, b_h_r2u, w_o_u2r, w_o_r2u)


def _final_body(au2_ref, ar2_ref, du_ref, dr_ref, bou_ref, bor_ref,
                nu_ref, nr_ref):
  normu = _norm_col(du_ref[:, 0:1])
  normr = _norm_col(dr_ref[:, 0:1])
  ou = au2_ref[...] * normu + bor_ref[...]
  nu_ref[...] = ou / jnp.maximum(jnp.sqrt(jnp.sum(ou * ou, axis=1, keepdims=True)), 1e-12)
  orr = ar2_ref[...] * normr + bou_ref[...]
  nr_ref[...] = orr / jnp.maximum(jnp.sqrt(jnp.sum(orr * orr, axis=1, keepdims=True)), 1e-12)


def _tc_final(aggu2, aggr2, degu, degr, b_o_u2r, b_o_r2u):
  full = jax.ShapeDtypeStruct((_N_U, _D3), jnp.float32)
  return pl.pallas_call(
      _final_body,
      grid=(_GRID_N,),
      in_specs=[
          _rows((_RB, _D3)), _rows((_RB, _D3)),
          _rows((_RB, _L)), _rows((_RB, _L)),
          _full((1, _D3)), _full((1, _D3)),
      ],
      out_specs=[_rows((_RB, _D3))] * 2,
      out_shape=(full, full),
  )(aggu2, aggr2, degu, degr, b_o_u2r, b_o_r2u)


_SB = _SPAD // 16


def _dots_body(ap_ref, bp_ref, an_ref, bn_ref, p_ref, n_ref):
  p_ref[...] = jnp.sum(ap_ref[...] * bp_ref[...], axis=1, keepdims=True)
  n_ref[...] = jnp.sum(an_ref[...] * bn_ref[...], axis=1, keepdims=True)


def _tc_dots(gap, gbp, gan, gbn):
  out = jax.ShapeDtypeStruct((_SPAD, 1), jnp.float32)
  return pl.pallas_call(
      _dots_body,
      grid=(_SPAD // _SB,),
      in_specs=[_rows((_SB, _D3))] * 4,
      out_specs=[_rows((_SB, 1))] * 2,
      out_shape=(out, out),
  )(gap, gbp, gan, gbn)


def kernel(user_feat, repo_feat, edge_src, edge_dst, pos_src, pos_dst,
           neg_src, neg_dst, W_user, b_user, W_repo, b_repo,
           W_h_u2r, b_h_u2r, W_h_r2u, b_h_r2u,
           W_o_u2r, b_o_u2r, W_o_r2u, b_o_r2u):
  epad = _EPAD - _E
  zer = jnp.zeros((epad,), jnp.int32)
  dum = jnp.full((epad,), _DUMMY, jnp.int32)
  es_g = jnp.concatenate([edge_src, zer])
  es_s = jnp.concatenate([edge_src, dum])
  ed_g = jnp.concatenate([edge_dst, zer])
  ed_s = jnp.concatenate([edge_dst, dum])

  degu, degr = _sc_degrees(es_s, ed_s)

  tul, tuh, trl, trh = _tc_embed(
      user_feat, repo_feat, degu, degr,
      W_user, b_user.reshape(1, -1), W_repo, b_repo.reshape(1, -1),
      W_h_u2r, W_h_r2u)

  arl, arh, aul, auh = _sc_agg64(tul, tuh, trl, trh, es_g, es_s, ed_g, ed_s)

  qu, qr = _tc_mid(arl, arh, aul, auh, degu, degr,
                   b_h_u2r.reshape(1, -1), b_h_r2u.reshape(1, -1),
                   W_o_u2r, W_o_r2u)

  aggr2, aggu2 = _sc_agg32(qu, qr, es_g, es_s, ed_g, ed_s)

  nu, nr = _tc_final(aggu2, aggr2, degu, degr,
                     b_o_u2r.reshape(1, -1), b_o_r2u.reshape(1, -1))

  spad = _SPAD - _EP
  zpad = jnp.zeros((spad,), jnp.int32)
  ps = jnp.concatenate([pos_src, zpad])
  pd = jnp.concatenate([pos_dst, zpad])
  ns_ = jnp.concatenate([neg_src, zpad])
  nd = jnp.concatenate([neg_dst, zpad])

  gap, gbp, gan, gbn = _sc_score_gather(nu, nr, ps, pd, ns_, nd)
  pos, neg = _tc_dots(gap, gbp, gan, gbn)
  return pos[:_EP], neg[:_EP]


# speedup vs baseline: 7.0051x; 1.0022x over previous
import functools
import jax
import jax.numpy as jnp
from jax import lax
from jax.experimental import pallas as pl
from jax.experimental.pallas import tpu as pltpu
from jax.experimental.pallas import tpu_sc as plsc

_N_U = 50000
_N_R = 50000
_E = 800000
_EP = 100000
_D1, _D2, _D3 = 64, 64, 32
_NC, _NS, _L = 2, 16, 16
_CH = 128
_NP = 50176
_RPT = _NP // _NS
_ZR = 224
_DUMMY = _NP - 1

_EPAD = 802816
_NBE = 392
_SPAD = 106496
_NBS = 52
_R = 4

_SC_MESH = plsc.VectorSubcoreMesh(core_axis_name="c", subcore_axis_name="s")
_SC_PARAMS = pltpu.CompilerParams(use_tc_tiling_on_sc=False)


def _fill_rows(buf, nrows, ncols, value):
  def row(i, _):
    for c0 in range(0, ncols, _L):
      buf[i, c0:c0 + _L] = jnp.full((_L,), value, jnp.float32)
    return _
  lax.fori_loop(0, nrows, row, None)


def _zero_own_rows(agg_sh, zbuf, tid):
  r0 = tid * _RPT
  def blk(k, _):
    pltpu.sync_copy(zbuf, agg_sh.at[pl.ds(r0 + k * _ZR, _ZR)])
    return _
  lax.fori_loop(0, _RPT // _ZR, blk, None)


def _flush_own_rows(agg_sh, out_hbm, tid):
  r0 = tid * _RPT
  pltpu.sync_copy(agg_sh.at[pl.ds(r0, _RPT)], out_hbm.at[pl.ds(r0, _RPT)])


def _chunk_base(tid, j):
  return (tid + _NS * j) * _CH


def _run_pipeline(nb, step, prologue):
  prologue()
  for r in range(_R):
    step(r, r, r < 2, True, True)
  def body(jr, _):
    j = _R * jr
    for r in range(_R):
      step(j + r, r, False, True, True)
    return _
  lax.fori_loop(1, nb // _R - 1, body, None)
  jb = nb - _R
  for r in range(_R):
    j = jb + r
    step(j, r, False, j + 2 < nb, j + 1 < nb)


def _agg_direction(tid, tab, gih, sih, out_hbm, bufs, zbuf, agg_sh):
  ig, is_, rows, si, sg, ss = bufs

  def start_idx(j, r):
    base = _chunk_base(tid, j)
    pltpu.async_copy(gih.at[pl.ds(base, _CH)], ig[r], si[r])
    pltpu.async_copy(sih.at[pl.ds(base, _CH)], is_[r], si[r])

  def wait_idx(r):
    pltpu.make_async_copy(gih.at[pl.ds(0, _CH)], ig[r], si[r]).wait()
    pltpu.make_async_copy(sih.at[pl.ds(0, _CH)], is_[r], si[r]).wait()

  def step(j, r, first, start2, gath1):
    r1, r2 = (r + 1) % _R, (r + 2) % _R
    if not first:
      pltpu.make_async_copy(rows[r2], agg_sh.at[is_[r2]], ss[r2]).wait()
    if start2:
      start_idx(j + 2, r2)
    if gath1:
      wait_idx(r1)
      pltpu.async_copy(tab.at[ig[r1]], rows[r1], sg[r1])
    pltpu.make_async_copy(tab.at[ig[r]], rows[r], sg[r]).wait()
    pltpu.async_copy(rows[r], agg_sh.at[is_[r]], ss[r], add=True)

  def prologue():
    start_idx(0, 0)
    start_idx(1, 1)
    wait_idx(0)
    pltpu.async_copy(tab.at[ig[0]], rows[0], sg[0])

  _zero_own_rows(agg_sh, zbuf, tid)
  plsc.subcore_barrier()
  _run_pipeline(_NBE, step, prologue)
  for rr in ((_NBE - 2) % _R, (_NBE - 1) % _R):
    pltpu.make_async_copy(rows[rr], agg_sh.at[is_[rr]], ss[rr]).wait()
  plsc.subcore_barrier()
  _flush_own_rows(agg_sh, out_hbm, tid)


def _deg_kernel(srcs_hbm, dsts_hbm, degu_hbm, degr_hbm,
                i0, i1, i2, i3, ones_v, zbuf, agg_sh,
                s0, s1, s2, s3, ss0, ss1, ss2, ss3):
  core = lax.axis_index("c")
  tid = lax.axis_index("s")
  idx = (i0, i1, i2, i3)
  si = (s0, s1, s2, s3)
  ss = (ss0, ss1, ss2, ss3)
  _fill_rows(ones_v, _CH, _L, 1.0)
  _fill_rows(zbuf, _ZR, _L, 0.0)
  _zero_own_rows(agg_sh, zbuf, tid)
  plsc.subcore_barrier()

  def count(eh, outh):
    def step(j, r, first, start2, gath1):
      del gath1
      r2 = (r + 2) % _R
      if not first:
        pltpu.make_async_copy(ones_v, agg_sh.at[idx[r2]], ss[r2]).wait()
      if start2:
        base = _chunk_base(tid, j + 2)
        pltpu.async_copy(eh.at[pl.ds(base, _CH)], idx[r2], si[r2])
      pltpu.make_async_copy(eh.at[pl.ds(0, _CH)], idx[r], si[r]).wait()
      pltpu.async_copy(ones_v, agg_sh.at[idx[r]], ss[r], add=True)

    def prologue():
      pltpu.async_copy(eh.at[pl.ds(_chunk_base(tid, 0), _CH)], idx[0], si[0])
      pltpu.async_copy(eh.at[pl.ds(_chunk_base(tid, 1), _CH)], idx[1], si[1])

    _run_pipeline(_NBE, step, prologue)
    for rr in ((_NBE - 2) % _R, (_NBE - 1) % _R):
      pltpu.make_async_copy(ones_v, agg_sh.at[idx[rr]], ss[rr]).wait()
    plsc.subcore_barrier()
    _flush_own_rows(agg_sh, outh, tid)

  @pl.when(core == 0)
  def _():
    count(srcs_hbm, degu_hbm)

  @pl.when(core == 1)
  def _():
    count(dsts_hbm, degr_hbm)


def _sc_degrees(es_s, ed_s):
  out = (jax.ShapeDtypeStruct((_NP, _L), jnp.float32),
         jax.ShapeDtypeStruct((_NP, _L), jnp.float32))
  return pl.kernel(
      _deg_kernel,
      out_type=out,
      mesh=_SC_MESH,
      compiler_params=_SC_PARAMS,
      scratch_types=[pltpu.VMEM((_CH,), jnp.int32)] * 4 + [
          pltpu.VMEM((_CH, _L), jnp.float32),
          pltpu.VMEM((_ZR, _L), jnp.float32),
          pltpu.VMEM_SHARED((_NP, _L), jnp.float32),
      ] + [pltpu.SemaphoreType.DMA] * 8,
  )(es_s, ed_s)


def _make_agg(d):
  def body(tlo, thi, gih, sih, out_lo, out_hi, *sc):
    core = lax.axis_index("c")
    tid = lax.axis_index("s")
    (g0, g1, g2, g3, x0, x1, x2, x3, r0, r1, r2, r3, zbuf, agg_sh,
     a0, a1, a2, a3, b0, b1, b2, b3, c0, c1, c2, c3) = sc
    bufs = ((g0, g1, g2, g3), (x0, x1, x2, x3), (r0, r1, r2, r3),
            (a0, a1, a2, a3), (b0, b1, b2, b3), (c0, c1, c2, c3))
    _fill_rows(zbuf, _ZR, d, 0.0)

    @pl.when(core == 0)
    def _():
      _agg_direction(tid, tlo, gih, sih, out_lo, bufs, zbuf, agg_sh)

    @pl.when(core == 1)
    def _():
      _agg_direction(tid, thi, gih, sih, out_hi, bufs, zbuf, agg_sh)

  half = jax.ShapeDtypeStruct((_NP, d), jnp.float32)

  def call(tlo, thi, gih, sih):
    return pl.kernel(
        body,
        out_type=(half, half),
        mesh=_SC_MESH,
        compiler_params=_SC_PARAMS,
        scratch_types=[pltpu.VMEM((_CH,), jnp.int32)] * 8 +
                      [pltpu.VMEM((_CH, d), jnp.float32)] * 4 +
                      [pltpu.VMEM((_ZR, d), jnp.float32),
                       pltpu.VMEM_SHARED((_NP, d), jnp.float32)] +
                      [pltpu.SemaphoreType.DMA] * 12,
    )(tlo, thi, gih, sih)

  return call


_agg_half32 = _make_agg(32)
_agg_half16 = _make_agg(16)


def _score2_kernel(tab, ia, ib, oa, ob, *sc):
  core = lax.axis_index("c")
  tid = lax.axis_index("s")
  (g0, g1, g2, g3, r0, r1, r2, r3,
   a0, a1, a2, a3, b0, b1, b2, b3, c0, c1, c2, c3) = sc
  ig = (g0, g1, g2, g3)
  rows = (r0, r1, r2, r3)
  si = (a0, a1, a2, a3)
  sg = (b0, b1, b2, b3)
  so = (c0, c1, c2, c3)

  def gth(idx_hbm, out_hbm):
    def step(j, r, first, start2, gath1):
      r1, r2 = (r + 1) % _R, (r + 2) % _R
      if not first:
        pltpu.make_async_copy(rows[r2], out_hbm.at[pl.ds(0, _CH)], so[r2]).wait()
      if start2:
        base = _chunk_base(tid, j + 2)
        pltpu.async_copy(idx_hbm.at[pl.ds(base, _CH)], ig[r2], si[r2])
      if gath1:
        pltpu.make_async_copy(idx_hbm.at[pl.ds(0, _CH)], ig[r1], si[r1]).wait()
        pltpu.async_copy(tab.at[ig[r1]], rows[r1], sg[r1])
      pltpu.make_async_copy(tab.at[ig[r]], rows[r], sg[r]).wait()
      pltpu.async_copy(rows[r], out_hbm.at[pl.ds(_chunk_base(tid, j), _CH)], so[r])

    def prologue():
      pltpu.async_copy(idx_hbm.at[pl.ds(_chunk_base(tid, 0), _CH)], ig[0], si[0])
      pltpu.async_copy(idx_hbm.at[pl.ds(_chunk_base(tid, 1), _CH)], ig[1], si[1])
      pltpu.make_async_copy(idx_hbm.at[pl.ds(0, _CH)], ig[0], si[0]).wait()
      pltpu.async_copy(tab.at[ig[0]], rows[0], sg[0])

    _run_pipeline(_NBS, step, prologue)
    for rr in ((_NBS - 2) % _R, (_NBS - 1) % _R):
      pltpu.make_async_copy(rows[rr], out_hbm.at[pl.ds(0, _CH)], so[rr]).wait()

  @pl.when(core == 0)
  def _():
    gth(ia, oa)

  @pl.when(core == 1)
  def _():
    gth(ib, ob)


def _sc_score2(tab, ia, ib):
  g = jax.ShapeDtypeStruct((_SPAD, _D3), jnp.float32)
  return pl.kernel(
      _score2_kernel,
      out_type=(g, g),
      mesh=_SC_MESH,
      compiler_params=_SC_PARAMS,
      scratch_types=[pltpu.VMEM((_CH,), jnp.int32)] * 4 +
                    [pltpu.VMEM((_CH, _D3), jnp.float32)] * 4 +
                    [pltpu.SemaphoreType.DMA] * 12,
  )(tab, ia, ib)


_RB = 2000
_GRID_N = _N_U // _RB


def _norm_col(deg_col):
  return jnp.where(deg_col > 0, lax.rsqrt(jnp.maximum(deg_col, 1.0)), 0.0)


def _full(shape):
  return pl.BlockSpec(shape, lambda i: tuple(0 for _ in shape))


def _rows(shape):
  return pl.BlockSpec(shape, lambda i: (i,) + tuple(0 for _ in shape[1:]))


def _embed1_body(u_ref, r_ref, wu_ref, bu_ref, wr_ref, br_ref,
                 whu_ref, whr_ref, hu2_ref, hr2_ref):
  hu = jnp.dot(u_ref[...], wu_ref[...], preferred_element_type=jnp.float32) + bu_ref[...]
  hu2_ref[...] = jnp.dot(hu, whu_ref[...], preferred_element_type=jnp.float32)
  hr = jnp.dot(r_ref[...], wr_ref[...], preferred_element_type=jnp.float32) + br_ref[...]
  hr2_ref[...] = jnp.dot(hr, whr_ref[...], preferred_element_type=jnp.float32)


def _tc_embed1(user_feat, repo_feat, w_user, b_user, w_repo, b_repo,
               w_h_u2r, w_h_r2u):
  full = jax.ShapeDtypeStruct((_N_U, _D2), jnp.float32)
  return pl.pallas_call(
      _embed1_body,
      grid=(_GRID_N,),
      in_specs=[
          _rows((_RB, 128)), _rows((_RB, 128)),
          _full((128, _D1)), _full((1, _D1)),
          _full((128, _D1)), _full((1, _D1)),
          _full((_D1, _D2)), _full((_D1, _D2)),
      ],
      out_specs=[_rows((_RB, _D2))] * 2,
      out_shape=(full, full),
  )(user_feat, repo_feat, w_user, b_user, w_repo, b_repo, w_h_u2r, w_h_r2u)


def _scale_body(hu2_ref, hr2_ref, du_ref, dr_ref,
                tul_ref, tuh_ref, trl_ref, trh_ref):
  pu = hu2_ref[...] * _norm_col(du_ref[:, 0:1])
  tul_ref[...] = pu[:, :_D3]
  tuh_ref[...] = pu[:, _D3:]
  pr = hr2_ref[...] * _norm_col(dr_ref[:, 0:1])
  trl_ref[...] = pr[:, :_D3]
  trh_ref[...] = pr[:, _D3:]


def _tc_scale(hu2, hr2, degu, degr):
  half = jax.ShapeDtypeStruct((_N_U, _D3), jnp.float32)
  return pl.pallas_call(
      _scale_body,
      grid=(_GRID_N,),
      in_specs=[
          _rows((_RB, _D2)), _rows((_RB, _D2)),
          _rows((_RB, _L)), _rows((_RB, _L)),
      ],
      out_specs=[_rows((_RB, _D3))] * 4,
      out_shape=(half, half, half, half),
  )(hu2, hr2, degu, degr)


def _mid_body(alo_ref, ahi_ref, d_ref, b_ref, w_ref, qlo_ref, qhi_ref):
  norm = _norm_col(d_ref[:, 0:1])
  h = jnp.concatenate([alo_ref[...], ahi_ref[...]], axis=1) * norm + b_ref[...]
  q = jnp.dot(h, w_ref[...], preferred_element_type=jnp.float32) * norm
  qlo_ref[...] = q[:, :_D3 // 2]
  qhi_ref[...] = q[:, _D3 // 2:]


def _tc_mid_dir(alo, ahi, deg, b_h, w_o):
  half = jax.ShapeDtypeStruct((_N_U, _D3 // 2), jnp.float32)
  return pl.pallas_call(
      _mid_body,
      grid=(_GRID_N,),
      in_specs=[
          _rows((_RB, _D3)), _rows((_RB, _D3)), _rows((_RB, _L)),
          _full((1, _D2)), _full((_D2, _D3)),
      ],
      out_specs=[_rows((_RB, _D3 // 2))] * 2,
      out_shape=(half, half),
  )(alo, ahi, deg, b_h, w_o)


def _final_body(alo_ref, ahi_ref, d_ref, b_ref, n_ref):
  norm = _norm_col(d_ref[:, 0:1])
  o = jnp.concatenate([alo_ref[...], ahi_ref[...]], axis=1) * norm + b_ref[...]
  n_ref[...] = o / jnp.maximum(jnp.sqrt(jnp.sum(o * o, axis=1, keepdims=True)), 1e-12)


def _tc_final_dir(alo, ahi, deg, b_o):
  full = jax.ShapeDtypeStruct((_N_U, _D3), jnp.float32)
  return pl.pallas_call(
      _final_body,
      grid=(_GRID_N,),
      in_specs=[
          _rows((_RB, _D3 // 2)), _rows((_RB, _D3 // 2)), _rows((_RB, _L)),
          _full((1, _D3)),
      ],
      out_specs=[_rows((_RB, _D3))],
      out_shape=(full,),
  )(alo, ahi, deg, b_o)[0]


_SB = _SPAD // 16


def _dots_body(ap_ref, bp_ref, an_ref, bn_ref, p_ref, n_ref):
  p_ref[...] = jnp.sum(ap_ref[...] * bp_ref[...], axis=1, keepdims=True)
  n_ref[...] = jnp.sum(an_ref[...] * bn_ref[...], axis=1, keepdims=True)


def _tc_dots(gap, gbp, gan, gbn):
  out = jax.ShapeDtypeStruct((_SPAD, 1), jnp.float32)
  return pl.pallas_call(
      _dots_body,
      grid=(_SPAD // _SB,),
      in_specs=[_rows((_SB, _D3))] * 4,
      out_specs=[_rows((_SB, 1))] * 2,
      out_shape=(out, out),
  )(gap, gbp, gan, gbn)


def kernel(user_feat, repo_feat, edge_src, edge_dst, pos_src, pos_dst,
           neg_src, neg_dst, W_user, b_user, W_repo, b_repo,
           W_h_u2r, b_h_u2r, W_h_r2u, b_h_r2u,
           W_o_u2r, b_o_u2r, W_o_r2u, b_o_r2u):
  epad = _EPAD - _E
  zer = jnp.zeros((epad,), jnp.int32)
  dum = jnp.full((epad,), _DUMMY, jnp.int32)
  es_g = jnp.concatenate([edge_src, zer])
  es_s = jnp.concatenate([edge_src, dum])
  ed_g = jnp.concatenate([edge_dst, zer])
  ed_s = jnp.concatenate([edge_dst, dum])

  degu, degr = _sc_degrees(es_s, ed_s)
  hu2, hr2 = _tc_embed1(user_feat, repo_feat,
                        W_user, b_user.reshape(1, -1),
                        W_repo, b_repo.reshape(1, -1), W_h_u2r, W_h_r2u)
  tul, tuh, trl, trh = _tc_scale(hu2, hr2, degu, degr)

  arl, arh = _agg_half32(tul, tuh, es_g, ed_s)
  aul, auh = _agg_half32(trl, trh, ed_g, es_s)

  qr_lo, qr_hi = _tc_mid_dir(arl, arh, degr, b_h_u2r.reshape(1, -1), W_o_r2u)
  qu_lo, qu_hi = _tc_mid_dir(aul, auh, degu, b_h_r2u.reshape(1, -1), W_o_u2r)

  au2_lo, au2_hi = _agg_half16(qr_lo, qr_hi, ed_g, es_s)
  ar2_lo, ar2_hi = _agg_half16(qu_lo, qu_hi, es_g, ed_s)

  nu = _tc_final_dir(au2_lo, au2_hi, degu, b_o_r2u.reshape(1, -1))
  nr = _tc_final_dir(ar2_lo, ar2_hi, degr, b_o_u2r.reshape(1, -1))

  spad = _SPAD - _EP
  zpad = jnp.zeros((spad,), jnp.int32)
  ps = jnp.concatenate([pos_src, zpad])
  pd = jnp.concatenate([pos_dst, zpad])
  ns_ = jnp.concatenate([neg_src, zpad])
  nd = jnp.concatenate([neg_dst, zpad])

  gap, gan = _sc_score2(nu, ps, ns_)
  gbp, gbn = _sc_score2(nr, pd, nd)
  pos, neg = _tc_dots(gap, gbp, gan, gbn)
  return pos[:_EP], neg[:_EP]


# speedup vs baseline: 7.3545x; 1.0499x over previous
import functools
import jax
import jax.numpy as jnp
from jax import lax
from jax.experimental import pallas as pl
from jax.experimental.pallas import tpu as pltpu
from jax.experimental.pallas import tpu_sc as plsc

_N_U = 50000
_N_R = 50000
_E = 800000
_EP = 100000
_D1, _D2, _D3 = 64, 64, 32
_NC, _NS, _L = 2, 16, 16
_CH = 128
_NP = 50176
_RPT = _NP // _NS
_ZR = 224
_DUMMY = _NP - 1

_EPAD = 802816
_NBE = 392
_SPAD = 106496
_NBS = 52
_R = 4

_SC_MESH = plsc.VectorSubcoreMesh(core_axis_name="c", subcore_axis_name="s")
_SC_PARAMS = pltpu.CompilerParams(use_tc_tiling_on_sc=False)


def _fill_rows(buf, nrows, ncols, value):
  def row(i, _):
    for c0 in range(0, ncols, _L):
      buf[i, c0:c0 + _L] = jnp.full((_L,), value, jnp.float32)
    return _
  lax.fori_loop(0, nrows, row, None)


def _zero_own_rows(agg_sh, zbuf, tid):
  r0 = tid * _RPT
  def blk(k, _):
    pltpu.sync_copy(zbuf, agg_sh.at[pl.ds(r0 + k * _ZR, _ZR)])
    return _
  lax.fori_loop(0, _RPT // _ZR, blk, None)


def _flush_own_rows(agg_sh, out_hbm, tid):
  r0 = tid * _RPT
  pltpu.sync_copy(agg_sh.at[pl.ds(r0, _RPT)], out_hbm.at[pl.ds(r0, _RPT)])


def _chunk_base(tid, j):
  return (tid + _NS * j) * _CH


def _run_pipeline(nb, step, prologue):
  prologue()
  for r in range(_R):
    step(r, r, r < 2, True, True)
  def body(jr, _):
    j = _R * jr
    for r in range(_R):
      step(j + r, r, False, True, True)
    return _
  lax.fori_loop(1, nb // _R - 1, body, None)
  jb = nb - _R
  for r in range(_R):
    j = jb + r
    step(j, r, False, j + 2 < nb, j + 1 < nb)


def _agg_direction(tid, tab, gih, sih, out_hbm, bufs, zbuf, agg_sh):
  ig, is_, rows, si, sg, ss = bufs

  def start_idx(j, r):
    base = _chunk_base(tid, j)
    pltpu.async_copy(gih.at[pl.ds(base, _CH)], ig[r], si[r])
    pltpu.async_copy(sih.at[pl.ds(base, _CH)], is_[r], si[r])

  def wait_idx(r):
    pltpu.make_async_copy(gih.at[pl.ds(0, _CH)], ig[r], si[r]).wait()
    pltpu.make_async_copy(sih.at[pl.ds(0, _CH)], is_[r], si[r]).wait()

  def step(j, r, first, start2, gath1):
    r1, r2 = (r + 1) % _R, (r + 2) % _R
    if not first:
      pltpu.make_async_copy(rows[r2], agg_sh.at[is_[r2]], ss[r2]).wait()
    if start2:
      start_idx(j + 2, r2)
    if gath1:
      wait_idx(r1)
      pltpu.async_copy(tab.at[ig[r1]], rows[r1], sg[r1])
    pltpu.make_async_copy(tab.at[ig[r]], rows[r], sg[r]).wait()
    pltpu.async_copy(rows[r], agg_sh.at[is_[r]], ss[r], add=True)

  def prologue():
    start_idx(0, 0)
    start_idx(1, 1)
    wait_idx(0)
    pltpu.async_copy(tab.at[ig[0]], rows[0], sg[0])

  _zero_own_rows(agg_sh, zbuf, tid)
  plsc.subcore_barrier()
  _run_pipeline(_NBE, step, prologue)
  for rr in ((_NBE - 2) % _R, (_NBE - 1) % _R):
    pltpu.make_async_copy(rows[rr], agg_sh.at[is_[rr]], ss[rr]).wait()
  plsc.subcore_barrier()
  _flush_own_rows(agg_sh, out_hbm, tid)


def _deg_kernel(srcs_hbm, dsts_hbm, degu_hbm, degr_hbm,
                i0, i1, i2, i3, ones_v, zbuf, agg_sh,
                s0, s1, s2, s3, ss0, ss1, ss2, ss3):
  core = lax.axis_index("c")
  tid = lax.axis_index("s")
  idx = (i0, i1, i2, i3)
  si = (s0, s1, s2, s3)
  ss = (ss0, ss1, ss2, ss3)
  _fill_rows(ones_v, _CH, _L, 1.0)
  _fill_rows(zbuf, _ZR, _L, 0.0)
  _zero_own_rows(agg_sh, zbuf, tid)
  plsc.subcore_barrier()

  def count(eh, outh):
    def step(j, r, first, start2, gath1):
      del gath1
      r2 = (r + 2) % _R
      if not first:
        pltpu.make_async_copy(ones_v, agg_sh.at[idx[r2]], ss[r2]).wait()
      if start2:
        base = _chunk_base(tid, j + 2)
        pltpu.async_copy(eh.at[pl.ds(base, _CH)], idx[r2], si[r2])
      pltpu.make_async_copy(eh.at[pl.ds(0, _CH)], idx[r], si[r]).wait()
      pltpu.async_copy(ones_v, agg_sh.at[idx[r]], ss[r], add=True)

    def prologue():
      pltpu.async_copy(eh.at[pl.ds(_chunk_base(tid, 0), _CH)], idx[0], si[0])
      pltpu.async_copy(eh.at[pl.ds(_chunk_base(tid, 1), _CH)], idx[1], si[1])

    _run_pipeline(_NBE, step, prologue)
    for rr in ((_NBE - 2) % _R, (_NBE - 1) % _R):
      pltpu.make_async_copy(ones_v, agg_sh.at[idx[rr]], ss[rr]).wait()
    plsc.subcore_barrier()
    _flush_own_rows(agg_sh, outh, tid)

  @pl.when(core == 0)
  def _():
    count(srcs_hbm, degu_hbm)

  @pl.when(core == 1)
  def _():
    count(dsts_hbm, degr_hbm)


def _sc_degrees(es_s, ed_s):
  out = (jax.ShapeDtypeStruct((_NP, _L), jnp.float32),
         jax.ShapeDtypeStruct((_NP, _L), jnp.float32))
  return pl.kernel(
      _deg_kernel,
      out_type=out,
      mesh=_SC_MESH,
      compiler_params=_SC_PARAMS,
      scratch_types=[pltpu.VMEM((_CH,), jnp.int32)] * 4 + [
          pltpu.VMEM((_CH, _L), jnp.float32),
          pltpu.VMEM((_ZR, _L), jnp.float32),
          pltpu.VMEM_SHARED((_NP, _L), jnp.float32),
      ] + [pltpu.SemaphoreType.DMA] * 8,
  )(es_s, ed_s)


def _make_agg(d):
  def body(tlo, thi, gih, sih, out_lo, out_hi, *sc):
    core = lax.axis_index("c")
    tid = lax.axis_index("s")
    (g0, g1, g2, g3, x0, x1, x2, x3, r0, r1, r2, r3, zbuf, agg_sh,
     a0, a1, a2, a3, b0, b1, b2, b3, c0, c1, c2, c3) = sc
    bufs = ((g0, g1, g2, g3), (x0, x1, x2, x3), (r0, r1, r2, r3),
            (a0, a1, a2, a3), (b0, b1, b2, b3), (c0, c1, c2, c3))
    _fill_rows(zbuf, _ZR, d, 0.0)

    @pl.when(core == 0)
    def _():
      _agg_direction(tid, tlo, gih, sih, out_lo, bufs, zbuf, agg_sh)

    @pl.when(core == 1)
    def _():
      _agg_direction(tid, thi, gih, sih, out_hi, bufs, zbuf, agg_sh)

  half = jax.ShapeDtypeStruct((_NP, d), jnp.float32)

  def call(tlo, thi, gih, sih):
    return pl.kernel(
        body,
        out_type=(half, half),
        mesh=_SC_MESH,
        compiler_params=_SC_PARAMS,
        scratch_types=[pltpu.VMEM((_CH,), jnp.int32)] * 8 +
                      [pltpu.VMEM((_CH, d), jnp.float32)] * 4 +
                      [pltpu.VMEM((_ZR, d), jnp.float32),
                       pltpu.VMEM_SHARED((_NP, d), jnp.float32)] +
                      [pltpu.SemaphoreType.DMA] * 12,
    )(tlo, thi, gih, sih)

  return call


_agg_half32 = _make_agg(32)
_agg_half16 = _make_agg(16)


def _agg32dir_kernel(qu, qr, es_g, es_s, ed_g, ed_s, aggr2, aggu2, *sc):
  core = lax.axis_index("c")
  tid = lax.axis_index("s")
  (g0, g1, g2, g3, x0, x1, x2, x3, r0, r1, r2, r3, zbuf, agg_sh,
   a0, a1, a2, a3, b0, b1, b2, b3, c0, c1, c2, c3) = sc
  bufs = ((g0, g1, g2, g3), (x0, x1, x2, x3), (r0, r1, r2, r3),
          (a0, a1, a2, a3), (b0, b1, b2, b3), (c0, c1, c2, c3))
  _fill_rows(zbuf, _ZR, _D3, 0.0)

  @pl.when(core == 0)
  def _():
    _agg_direction(tid, qu, es_g, ed_s, aggr2, bufs, zbuf, agg_sh)

  @pl.when(core == 1)
  def _():
    _agg_direction(tid, qr, ed_g, es_s, aggu2, bufs, zbuf, agg_sh)


def _sc_agg32dir(qu, qr, es_g, es_s, ed_g, ed_s):
  full = jax.ShapeDtypeStruct((_NP, _D3), jnp.float32)
  return pl.kernel(
      _agg32dir_kernel,
      out_type=(full, full),
      mesh=_SC_MESH,
      compiler_params=_SC_PARAMS,
      scratch_types=[pltpu.VMEM((_CH,), jnp.int32)] * 8 +
                    [pltpu.VMEM((_CH, _D3), jnp.float32)] * 4 +
                    [pltpu.VMEM((_ZR, _D3), jnp.float32),
                     pltpu.VMEM_SHARED((_NP, _D3), jnp.float32)] +
                    [pltpu.SemaphoreType.DMA] * 12,
  )(qu, qr, es_g, es_s, ed_g, ed_s)


def _score2_kernel(tab, ia, ib, oa, ob, *sc):
  core = lax.axis_index("c")
  tid = lax.axis_index("s")
  (g0, g1, g2, g3, r0, r1, r2, r3,
   a0, a1, a2, a3, b0, b1, b2, b3, c0, c1, c2, c3) = sc
  ig = (g0, g1, g2, g3)
  rows = (r0, r1, r2, r3)
  si = (a0, a1, a2, a3)
  sg = (b0, b1, b2, b3)
  so = (c0, c1, c2, c3)

  def gth(idx_hbm, out_hbm):
    def step(j, r, first, start2, gath1):
      r1, r2 = (r + 1) % _R, (r + 2) % _R
      if not first:
        pltpu.make_async_copy(rows[r2], out_hbm.at[pl.ds(0, _CH)], so[r2]).wait()
      if start2:
        base = _chunk_base(tid, j + 2)
        pltpu.async_copy(idx_hbm.at[pl.ds(base, _CH)], ig[r2], si[r2])
      if gath1:
        pltpu.make_async_copy(idx_hbm.at[pl.ds(0, _CH)], ig[r1], si[r1]).wait()
        pltpu.async_copy(tab.at[ig[r1]], rows[r1], sg[r1])
      pltpu.make_async_copy(tab.at[ig[r]], rows[r], sg[r]).wait()
      pltpu.async_copy(rows[r], out_hbm.at[pl.ds(_chunk_base(tid, j), _CH)], so[r])

    def prologue():
      pltpu.async_copy(idx_hbm.at[pl.ds(_chunk_base(tid, 0), _CH)], ig[0], si[0])
      pltpu.async_copy(idx_hbm.at[pl.ds(_chunk_base(tid, 1), _CH)], ig[1], si[1])
      pltpu.make_async_copy(idx_hbm.at[pl.ds(0, _CH)], ig[0], si[0]).wait()
      pltpu.async_copy(tab.at[ig[0]], rows[0], sg[0])

    _run_pipeline(_NBS, step, prologue)
    for rr in ((_NBS - 2) % _R, (_NBS - 1) % _R):
      pltpu.make_async_copy(rows[rr], out_hbm.at[pl.ds(0, _CH)], so[rr]).wait()

  @pl.when(core == 0)
  def _():
    gth(ia, oa)

  @pl.when(core == 1)
  def _():
    gth(ib, ob)


def _sc_score2(tab, ia, ib):
  g = jax.ShapeDtypeStruct((_SPAD, _D3), jnp.float32)
  return pl.kernel(
      _score2_kernel,
      out_type=(g, g),
      mesh=_SC_MESH,
      compiler_params=_SC_PARAMS,
      scratch_types=[pltpu.VMEM((_CH,), jnp.int32)] * 4 +
                    [pltpu.VMEM((_CH, _D3), jnp.float32)] * 4 +
                    [pltpu.SemaphoreType.DMA] * 12,
  )(tab, ia, ib)


_RB = 2000
_GRID_N = _N_U // _RB


def _norm_col(deg_col):
  return jnp.where(deg_col > 0, lax.rsqrt(jnp.maximum(deg_col, 1.0)), 0.0)


def _full(shape):
  return pl.BlockSpec(shape, lambda i: tuple(0 for _ in shape))


def _rows(shape):
  return pl.BlockSpec(shape, lambda i: (i,) + tuple(0 for _ in shape[1:]))


def _embed1_body(u_ref, r_ref, wu_ref, bu_ref, wr_ref, br_ref,
                 whu_ref, whr_ref, hu2_ref, hr2_ref):
  hu = jnp.dot(u_ref[...], wu_ref[...], preferred_element_type=jnp.float32) + bu_ref[...]
  hu2_ref[...] = jnp.dot(hu, whu_ref[...], preferred_element_type=jnp.float32)
  hr = jnp.dot(r_ref[...], wr_ref[...], preferred_element_type=jnp.float32) + br_ref[...]
  hr2_ref[...] = jnp.dot(hr, whr_ref[...], preferred_element_type=jnp.float32)


def _tc_embed1(user_feat, repo_feat, w_user, b_user, w_repo, b_repo,
               w_h_u2r, w_h_r2u):
  full = jax.ShapeDtypeStruct((_N_U, _D2), jnp.float32)
  return pl.pallas_call(
      _embed1_body,
      grid=(_GRID_N,),
      in_specs=[
          _rows((_RB, 128)), _rows((_RB, 128)),
          _full((128, _D1)), _full((1, _D1)),
          _full((128, _D1)), _full((1, _D1)),
          _full((_D1, _D2)), _full((_D1, _D2)),
      ],
      out_specs=[_rows((_RB, _D2))] * 2,
      out_shape=(full, full),
  )(user_feat, repo_feat, w_user, b_user, w_repo, b_repo, w_h_u2r, w_h_r2u)


def _scale_body(hu2_ref, hr2_ref, du_ref, dr_ref,
                tul_ref, tuh_ref, trl_ref, trh_ref):
  pu = hu2_ref[...] * _norm_col(du_ref[:, 0:1])
  tul_ref[...] = pu[:, :_D3]
  tuh_ref[...] = pu[:, _D3:]
  pr = hr2_ref[...] * _norm_col(dr_ref[:, 0:1])
  trl_ref[...] = pr[:, :_D3]
  trh_ref[...] = pr[:, _D3:]


def _tc_scale(hu2, hr2, degu, degr):
  half = jax.ShapeDtypeStruct((_N_U, _D3), jnp.float32)
  return pl.pallas_call(
      _scale_body,
      grid=(_GRID_N,),
      in_specs=[
          _rows((_RB, _D2)), _rows((_RB, _D2)),
          _rows((_RB, _L)), _rows((_RB, _L)),
      ],
      out_specs=[_rows((_RB, _D3))] * 4,
      out_shape=(half, half, half, half),
  )(hu2, hr2, degu, degr)


def _mid_body(alo_ref, ahi_ref, d_ref, b_ref, w_ref, q_ref):
  norm = _norm_col(d_ref[:, 0:1])
  h = jnp.concatenate([alo_ref[...], ahi_ref[...]], axis=1) * norm + b_ref[...]
  q_ref[...] = jnp.dot(h, w_ref[...], preferred_element_type=jnp.float32) * norm


def _tc_mid_dir(alo, ahi, deg, b_h, w_o):
  full = jax.ShapeDtypeStruct((_N_U, _D3), jnp.float32)
  return pl.pallas_call(
      _mid_body,
      grid=(_GRID_N,),
      in_specs=[
          _rows((_RB, _D3)), _rows((_RB, _D3)), _rows((_RB, _L)),
          _full((1, _D2)), _full((_D2, _D3)),
      ],
      out_specs=[_rows((_RB, _D3))],
      out_shape=(full,),
  )(alo, ahi, deg, b_h, w_o)[0]


def _final_body(a_ref, d_ref, b_ref, n_ref):
  norm = _norm_col(d_ref[:, 0:1])
  o = a_ref[...] * norm + b_ref[...]
  n_ref[...] = o / jnp.maximum(jnp.sqrt(jnp.sum(o * o, axis=1, keepdims=True)), 1e-12)


def _tc_final_dir(agg, deg, b_o):
  full = jax.ShapeDtypeStruct((_N_U, _D3), jnp.float32)
  return pl.pallas_call(
      _final_body,
      grid=(_GRID_N,),
      in_specs=[
          _rows((_RB, _D3)), _rows((_RB, _L)),
          _full((1, _D3)),
      ],
      out_specs=[_rows((_RB, _D3))],
      out_shape=(full,),
  )(agg, deg, b_o)[0]


_SB = _SPAD // 16


def _dots_body(ap_ref, bp_ref, an_ref, bn_ref, p_ref, n_ref):
  p_ref[...] = jnp.sum(ap_ref[...] * bp_ref[...], axis=1, keepdims=True)
  n_ref[...] = jnp.sum(an_ref[...] * bn_ref[...], axis=1, keepdims=True)


def _tc_dots(gap, gbp, gan, gbn):
  out = jax.ShapeDtypeStruct((_SPAD, 1), jnp.float32)
  return pl.pallas_call(
      _dots_body,
      grid=(_SPAD // _SB,),
      in_specs=[_rows((_SB, _D3))] * 4,
      out_specs=[_rows((_SB, 1))] * 2,
      out_shape=(out, out),
  )(gap, gbp, gan, gbn)


def kernel(user_feat, repo_feat, edge_src, edge_dst, pos_src, pos_dst,
           neg_src, neg_dst, W_user, b_user, W_repo, b_repo,
           W_h_u2r, b_h_u2r, W_h_r2u, b_h_r2u,
           W_o_u2r, b_o_u2r, W_o_r2u, b_o_r2u):
  epad = _EPAD - _E
  zer = jnp.zeros((epad,), jnp.int32)
  dum = jnp.full((epad,), _DUMMY, jnp.int32)
  es_g = jnp.concatenate([edge_src, zer])
  es_s = jnp.concatenate([edge_src, dum])
  ed_g = jnp.concatenate([edge_dst, zer])
  ed_s = jnp.concatenate([edge_dst, dum])

  degu, degr = _sc_degrees(es_s, ed_s)
  hu2, hr2 = _tc_embed1(user_feat, repo_feat,
                        W_user, b_user.reshape(1, -1),
                        W_repo, b_repo.reshape(1, -1), W_h_u2r, W_h_r2u)
  tul, tuh, trl, trh = _tc_scale(hu2, hr2, degu, degr)

  arl, arh = _agg_half32(tul, tuh, es_g, ed_s)
  aul, auh = _agg_half32(trl, trh, ed_g, es_s)

  qr = _tc_mid_dir(arl, arh, degr, b_h_u2r.reshape(1, -1), W_o_r2u)
  qu = _tc_mid_dir(aul, auh, degu, b_h_r2u.reshape(1, -1), W_o_u2r)

  aggr2, aggu2 = _sc_agg32dir(qu, qr, es_g, es_s, ed_g, ed_s)

  nu = _tc_final_dir(aggu2, degu, b_o_r2u.reshape(1, -1))
  nr = _tc_final_dir(aggr2, degr, b_o_u2r.reshape(1, -1))

  spad = _SPAD - _EP
  zpad = jnp.zeros((spad,), jnp.int32)
  ps = jnp.concatenate([pos_src, zpad])
  pd = jnp.concatenate([pos_dst, zpad])
  ns_ = jnp.concatenate([neg_src, zpad])
  nd = jnp.concatenate([neg_dst, zpad])

  gap, gan = _sc_score2(nu, ps, ns_)
  gbp, gbn = _sc_score2(nr, pd, nd)
  pos, neg = _tc_dots(gap, gbp, gan, gbn)
  return pos[:_EP], neg[:_EP]
